# baseline (speedup 1.0000x reference)
import functools

import jax
import jax.numpy as jnp
from jax import lax
from jax.experimental import pallas as pl
from jax.experimental.pallas import tpu as pltpu
from jax.experimental.pallas import tpu_sc as plsc

E = 8
K = 2
H = 1024
F = 4096
N = 4096
NK = N * K
CAP = int(1.25 * N * K / E) + 1
CB = 128
CPAD = 1408
ROWS = E * CPAD
FB = 512

NCORES = 2
NSUB = 16
NW = NCORES * NSUB


def _router_body(x_ref, rw_ref, pos_ref, cw_ref, cnt_ref):
    x = x_ref[...]
    rw = rw_ref[...]
    logits = lax.dot_general(
        x, rw, (((1,), (1,)), ((), ())),
        preferred_element_type=jnp.float32)

    ilane = lax.broadcasted_iota(jnp.int32, (N, E), 1)
    m0 = jnp.max(logits, axis=1, keepdims=True)
    e0 = jnp.min(jnp.where(logits == m0, ilane, E), axis=1, keepdims=True)
    b0 = ilane == e0
    p2 = jnp.where(b0, -jnp.inf, logits)
    m1 = jnp.max(p2, axis=1, keepdims=True)
    e1 = jnp.min(jnp.where(p2 == m1, ilane, E), axis=1, keepdims=True)
    b1 = ilane == e1

    b0f = b0.astype(jnp.float32)
    b1f = b1.astype(jnp.float32)
    s = b0f + b1f

    r128 = lax.broadcasted_iota(jnp.int32, (128, 128), 0)
    c128 = lax.broadcasted_iota(jnp.int32, (128, 128), 1)
    tri = (c128 < r128).astype(jnp.float32)
    parts = []
    carry = jnp.zeros((1, E), jnp.float32)
    for b in range(N // 128):
        blk = s[b * 128:(b + 1) * 128]
        inner = lax.dot_general(tri, blk, (((1,), (0,)), ((), ())),
                                preferred_element_type=jnp.float32)
        parts.append(inner + carry)
        carry = carry + jnp.sum(blk, axis=0, keepdims=True)
    excl = jnp.concatenate(parts, axis=0)

    rank0 = jnp.sum(excl * b0f, axis=1, keepdims=True).astype(jnp.int32)
    rank1 = jnp.sum(excl * b1f, axis=1, keepdims=True).astype(jnp.int32)
    keep0 = rank0 < CAP
    keep1 = rank1 < CAP

    trash = CPAD - 1
    pos0 = e0 * CPAD + jnp.where(keep0, rank0, trash)
    pos1 = e1 * CPAD + jnp.where(keep1, rank1, trash)
    w0 = 1.0 / (1.0 + jnp.exp(m1 - m0))
    cw0 = jnp.where(keep0, w0, 0.0)
    cw1 = jnp.where(keep1, 1.0 - w0, 0.0)

    pos_ref[...] = jnp.concatenate([pos0, pos1], axis=1)
    cw_ref[...] = jnp.concatenate([cw0, cw1], axis=1)
    kept = jnp.minimum(carry, float(CAP)).astype(jnp.int32)
    cnt_ref[...] = jnp.broadcast_to(kept, (8, E))


def _router_call(x_flat, router_w):
    return pl.pallas_call(
        _router_body,
        out_shape=(
            jax.ShapeDtypeStruct((N, K), jnp.int32),
            jax.ShapeDtypeStruct((N, K), jnp.float32),
            jax.ShapeDtypeStruct((8, E), jnp.int32),
        ),
    )(x_flat, router_w)


@functools.lru_cache(maxsize=1)
def _sc_mesh():
    return plsc.VectorSubcoreMesh(core_axis_name="c", subcore_axis_name="s")


_DCHUNK = 64
H32 = H // 2


def _dispatch_call(xb, posk):
    per_w = NK // NW

    @functools.partial(
        pl.kernel,
        out_type=jax.ShapeDtypeStruct((ROWS, H32), jnp.int32),
        mesh=_sc_mesh(),
        scratch_types=[
            pltpu.VMEM((_DCHUNK,), jnp.int32),
            pltpu.VMEM((_DCHUNK, H32), jnp.int32),
            pltpu.SemaphoreType.DMA,
        ],
    )
    def scatter_k(xb_hbm, pos_hbm, tok_hbm, idx_v, rows_v, sem):
        wid = lax.axis_index("s") * NCORES + lax.axis_index("c")
        base = wid * per_w

        @pl.loop(0, per_w, step=_DCHUNK)
        def _(off):
            s0 = base + off
            n0 = lax.rem(s0, N)
            pltpu.sync_copy(pos_hbm.at[pl.ds(s0, _DCHUNK)], idx_v)
            pltpu.sync_copy(xb_hbm.at[pl.ds(n0, _DCHUNK)], rows_v)
            pltpu.async_copy(rows_v, tok_hbm.at[idx_v], sem).wait()

    return scatter_k(xb, posk)


def _ffn_body(tok_ref, w1_ref, w2_ref, out_ref):
    xb = tok_ref[...]
    acc = jnp.zeros((CB, H), jnp.float32)
    for fb in range(F // FB):
        w1b = w1_ref[0, :, fb * FB:(fb + 1) * FB]
        h = lax.dot_general(xb, w1b, (((1,), (0,)), ((), ())),
                            preferred_element_type=jnp.float32)
        h = h / (1.0 + jnp.exp(-h))
        hb = h.astype(jnp.bfloat16)
        w2b = w2_ref[0, fb * FB:(fb + 1) * FB, :]
        acc = acc + lax.dot_general(hb, w2b, (((1,), (0,)), ((), ())),
                                    preferred_element_type=jnp.float32)
    out_ref[...] = acc


def _ffn_call(tok, w1b, w2b):
    nblk = CPAD // CB
    return pl.pallas_call(
        _ffn_body,
        grid=(E, nblk),
        in_specs=[
            pl.BlockSpec((CB, H), lambda e, cb: (e * nblk + cb, 0)),
            pl.BlockSpec((1, H, F), lambda e, cb: (e, 0, 0)),
            pl.BlockSpec((1, F, H), lambda e, cb: (e, 0, 0)),
        ],
        out_specs=pl.BlockSpec((CB, H), lambda e, cb: (e * nblk + cb, 0)),
        out_shape=jax.ShapeDtypeStruct((ROWS, H), jnp.float32),
    )(tok, w1b, w2b)


_GCHUNK = 32


def _gather_call(eo, posk):
    per_w = NK // NW

    @functools.partial(
        pl.kernel,
        out_type=jax.ShapeDtypeStruct((NK, H), jnp.float32),
        mesh=_sc_mesh(),
        scratch_types=[
            pltpu.VMEM((_GCHUNK,), jnp.int32),
            pltpu.VMEM((_GCHUNK, H), jnp.float32),
            pltpu.SemaphoreType.DMA,
        ],
    )
    def gather_k(eo_hbm, pos_hbm, g_hbm, idx_v, rows_v, sem):
        wid = lax.axis_index("s") * NCORES + lax.axis_index("c")
        base = wid * per_w

        @pl.loop(0, per_w, step=_GCHUNK)
        def _(off):
            s0 = base + off
            pltpu.sync_copy(pos_hbm.at[pl.ds(s0, _GCHUNK)], idx_v)
            pltpu.async_copy(eo_hbm.at[idx_v], rows_v, sem).wait()
            pltpu.sync_copy(rows_v, g_hbm.at[pl.ds(s0, _GCHUNK)])

    return gather_k(eo, posk)


_RB = 512


def _combine_body(x_ref, g0_ref, g1_ref, cw0_ref, cw1_ref, out_ref):
    cw0 = cw0_ref[...]
    cw1 = cw1_ref[...]
    t0 = jnp.where(cw0 > 0.0, g0_ref[...] * cw0, 0.0)
    t1 = jnp.where(cw1 > 0.0, g1_ref[...] * cw1, 0.0)
    out_ref[...] = x_ref[...] + t0 + t1


def _combine_call(x_flat, g, cw0, cw1):
    nblk = N // _RB
    return pl.pallas_call(
        _combine_body,
        grid=(nblk,),
        in_specs=[
            pl.BlockSpec((_RB, H), lambda i: (i, 0)),
            pl.BlockSpec((_RB, H), lambda i: (i, 0)),
            pl.BlockSpec((_RB, H), lambda i: (i + nblk, 0)),
            pl.BlockSpec((_RB, 1), lambda i: (i, 0)),
            pl.BlockSpec((_RB, 1), lambda i: (i, 0)),
        ],
        out_specs=pl.BlockSpec((_RB, H), lambda i: (i, 0)),
        out_shape=jax.ShapeDtypeStruct((N, H), jnp.float32),
    )(x_flat, g, g, cw0, cw1)


def kernel(x, router_w, w1, w2):
    B, T, _ = x.shape
    x_flat = x.reshape(N, H)
    w1b = w1.astype(jnp.bfloat16)
    w2b = w2.astype(jnp.bfloat16)

    pos_tok, cw_tok, _cnt = _router_call(x_flat, router_w)
    posk = pos_tok.T.reshape(NK)
    xb = x_flat.astype(jnp.bfloat16)
    xb32 = lax.bitcast_convert_type(xb.reshape(N, H32, 2), jnp.int32)

    tok32 = _dispatch_call(xb32, posk)
    tok = lax.bitcast_convert_type(tok32, jnp.bfloat16).reshape(ROWS, H)
    eo = _ffn_call(tok, w1b, w2b)
    g = _gather_call(eo, posk)
    out = _combine_call(x_flat, g, cw_tok[:, :1], cw_tok[:, 1:])
    return out.reshape(B, T, H)


# speedup vs baseline: 1.8216x; 1.8216x over previous
import functools

import jax
import jax.numpy as jnp
from jax import lax
from jax.experimental import pallas as pl
from jax.experimental.pallas import tpu as pltpu
from jax.experimental.pallas import tpu_sc as plsc

E = 8
K = 2
H = 1024
F = 4096
N = 4096
NK = N * K
CAP = int(1.25 * N * K / E) + 1
CB = 256
CPAD = 1536
ROWS = E * CPAD
FB = 512

NCORES = 2
NSUB = 16
NW = NCORES * NSUB


def _router_body(x_ref, rw_ref, pos0_ref, pos1_ref, cw0_ref, cw1_ref, cnt_ref):
    x = x_ref[...]
    rw = rw_ref[...]
    logits = lax.dot_general(
        x, rw, (((1,), (1,)), ((), ())),
        preferred_element_type=jnp.float32)

    ilane = lax.broadcasted_iota(jnp.int32, (N, E), 1)
    m0 = jnp.max(logits, axis=1, keepdims=True)
    e0 = jnp.min(jnp.where(logits == m0, ilane, E), axis=1, keepdims=True)
    b0 = ilane == e0
    p2 = jnp.where(b0, -jnp.inf, logits)
    m1 = jnp.max(p2, axis=1, keepdims=True)
    e1 = jnp.min(jnp.where(p2 == m1, ilane, E), axis=1, keepdims=True)
    b1 = ilane == e1

    b0f = b0.astype(jnp.float32)
    b1f = b1.astype(jnp.float32)
    s = b0f + b1f

    r128 = lax.broadcasted_iota(jnp.int32, (128, 128), 0)
    c128 = lax.broadcasted_iota(jnp.int32, (128, 128), 1)
    tri = (c128 < r128).astype(jnp.float32)
    parts = []
    carry = jnp.zeros((1, E), jnp.float32)
    for b in range(N // 128):
        blk = s[b * 128:(b + 1) * 128]
        inner = lax.dot_general(tri, blk, (((1,), (0,)), ((), ())),
                                preferred_element_type=jnp.float32)
        parts.append(inner + carry)
        carry = carry + jnp.sum(blk, axis=0, keepdims=True)
    excl = jnp.concatenate(parts, axis=0)

    rank0 = jnp.sum(excl * b0f, axis=1, keepdims=True).astype(jnp.int32)
    rank1 = jnp.sum(excl * b1f, axis=1, keepdims=True).astype(jnp.int32)
    keep0 = rank0 < CAP
    keep1 = rank1 < CAP

    trash = CPAD - 1
    pos0 = e0 * CPAD + jnp.where(keep0, rank0, trash)
    pos1 = e1 * CPAD + jnp.where(keep1, rank1, trash)
    w0 = 1.0 / (1.0 + jnp.exp(m1 - m0))
    cw0 = jnp.where(keep0, w0, 0.0)
    cw1 = jnp.where(keep1, 1.0 - w0, 0.0)

    pos0_ref[...] = pos0
    pos1_ref[...] = pos1
    cw0_ref[...] = cw0
    cw1_ref[...] = cw1
    kept = jnp.minimum(carry, float(CAP)).astype(jnp.int32)
    cnt_ref[...] = jnp.broadcast_to(kept, (8, E))


def _router_call(x_flat, router_w):
    return pl.pallas_call(
        _router_body,
        out_shape=(
            jax.ShapeDtypeStruct((N, 1), jnp.int32),
            jax.ShapeDtypeStruct((N, 1), jnp.int32),
            jax.ShapeDtypeStruct((N, 1), jnp.float32),
            jax.ShapeDtypeStruct((N, 1), jnp.float32),
            jax.ShapeDtypeStruct((8, E), jnp.int32),
        ),
    )(x_flat, router_w)


@functools.lru_cache(maxsize=1)
def _sc_mesh():
    return plsc.VectorSubcoreMesh(core_axis_name="c", subcore_axis_name="s")


_DCHUNK = 32


def _dispatch_call(xf, posk):
    per_w = NK // NW

    @functools.partial(
        pl.kernel,
        out_type=jax.ShapeDtypeStruct((ROWS, H), jnp.float32),
        mesh=_sc_mesh(),
        scratch_types=[
            pltpu.VMEM((_DCHUNK,), jnp.int32),
            pltpu.VMEM((_DCHUNK, H), jnp.float32),
            pltpu.SemaphoreType.DMA,
        ],
    )
    def scatter_k(xf_hbm, pos_hbm, tok_hbm, idx_v, rows_v, sem):
        wid = lax.axis_index("s") * NCORES + lax.axis_index("c")
        base = wid * per_w

        @pl.loop(0, per_w, step=_DCHUNK)
        def _(off):
            s0 = base + off
            n0 = lax.rem(s0, N)
            pltpu.sync_copy(pos_hbm.at[pl.ds(s0, _DCHUNK)], idx_v)
            pltpu.sync_copy(xf_hbm.at[pl.ds(n0, _DCHUNK)], rows_v)
            pltpu.async_copy(rows_v, tok_hbm.at[idx_v], sem).wait()

    return scatter_k(xf, posk)


def _ffn_body(kept_ref, tok_ref, w1_ref, w2_ref, out_ref):
    e = pl.program_id(0)
    cb = pl.program_id(1)

    @pl.when(kept_ref[e] > cb * CB)
    def _():
        xb = tok_ref[...].astype(jnp.bfloat16)
        acc = jnp.zeros((CB, H), jnp.float32)
        for fb in range(F // FB):
            w1b = w1_ref[0, :, fb * FB:(fb + 1) * FB]
            h = lax.dot_general(xb, w1b, (((1,), (0,)), ((), ())),
                                preferred_element_type=jnp.float32)
            h = h / (1.0 + jnp.exp(-h))
            hb = h.astype(jnp.bfloat16)
            w2b = w2_ref[0, fb * FB:(fb + 1) * FB, :]
            acc = acc + lax.dot_general(hb, w2b, (((1,), (0,)), ((), ())),
                                        preferred_element_type=jnp.float32)
        out_ref[...] = acc


def _ffn_call(kept, tok, w1b, w2b):
    nblk = CPAD // CB
    grid_spec = pltpu.PrefetchScalarGridSpec(
        num_scalar_prefetch=1,
        grid=(E, nblk),
        in_specs=[
            pl.BlockSpec((CB, H), lambda e, cb, k: (e * nblk + cb, 0)),
            pl.BlockSpec((1, H, F), lambda e, cb, k: (e, 0, 0)),
            pl.BlockSpec((1, F, H), lambda e, cb, k: (e, 0, 0)),
        ],
        out_specs=pl.BlockSpec((CB, H), lambda e, cb, k: (e * nblk + cb, 0)),
    )
    return pl.pallas_call(
        _ffn_body,
        grid_spec=grid_spec,
        out_shape=jax.ShapeDtypeStruct((ROWS, H), jnp.float32),
    )(kept, tok, w1b, w2b)


_GCHUNK = 32


def _gather_call(eo, posk):
    per_w = NK // NW

    @functools.partial(
        pl.kernel,
        out_type=jax.ShapeDtypeStruct((NK, H), jnp.float32),
        mesh=_sc_mesh(),
        scratch_types=[
            pltpu.VMEM((_GCHUNK,), jnp.int32),
            pltpu.VMEM((_GCHUNK, H), jnp.float32),
            pltpu.SemaphoreType.DMA,
        ],
    )
    def gather_k(eo_hbm, pos_hbm, g_hbm, idx_v, rows_v, sem):
        wid = lax.axis_index("s") * NCORES + lax.axis_index("c")
        base = wid * per_w

        @pl.loop(0, per_w, step=_GCHUNK)
        def _(off):
            s0 = base + off
            pltpu.sync_copy(pos_hbm.at[pl.ds(s0, _GCHUNK)], idx_v)
            pltpu.async_copy(eo_hbm.at[idx_v], rows_v, sem).wait()
            pltpu.sync_copy(rows_v, g_hbm.at[pl.ds(s0, _GCHUNK)])

    return gather_k(eo, posk)


_RB = 512


def _combine_body(x_ref, g0_ref, g1_ref, cw0_ref, cw1_ref, out_ref):
    cw0 = cw0_ref[...]
    cw1 = cw1_ref[...]
    t0 = jnp.where(cw0 > 0.0, g0_ref[...] * cw0, 0.0)
    t1 = jnp.where(cw1 > 0.0, g1_ref[...] * cw1, 0.0)
    out_ref[...] = x_ref[...] + t0 + t1


def _combine_call(x_flat, g, cw0, cw1):
    nblk = N // _RB
    return pl.pallas_call(
        _combine_body,
        grid=(nblk,),
        in_specs=[
            pl.BlockSpec((_RB, H), lambda i: (i, 0)),
            pl.BlockSpec((_RB, H), lambda i: (i, 0)),
            pl.BlockSpec((_RB, H), lambda i: (i + nblk, 0)),
            pl.BlockSpec((_RB, 1), lambda i: (i, 0)),
            pl.BlockSpec((_RB, 1), lambda i: (i, 0)),
        ],
        out_specs=pl.BlockSpec((_RB, H), lambda i: (i, 0)),
        out_shape=jax.ShapeDtypeStruct((N, H), jnp.float32),
    )(x_flat, g, g, cw0, cw1)


def kernel(x, router_w, w1, w2):
    B, T, _ = x.shape
    x_flat = x.reshape(N, H)
    w1b = w1.astype(jnp.bfloat16)
    w2b = w2.astype(jnp.bfloat16)

    pos0, pos1, cw0, cw1, cnt = _router_call(x_flat, router_w)
    posk = jnp.concatenate([pos0.reshape(N), pos1.reshape(N)])

    tok = _dispatch_call(x_flat, posk)
    eo = _ffn_call(cnt[0], tok, w1b, w2b)
    g = _gather_call(eo, posk)
    out = _combine_call(x_flat, g, cw0, cw1)
    return out.reshape(B, T, H)


# speedup vs baseline: 2.1858x; 1.1999x over previous
import functools

import jax
import jax.numpy as jnp
from jax import lax
from jax.experimental import pallas as pl
from jax.experimental.pallas import tpu as pltpu
from jax.experimental.pallas import tpu_sc as plsc

E = 8
K = 2
H = 1024
F = 4096
N = 4096
NK = N * K
CAP = int(1.25 * N * K / E) + 1
CB = 256
CPAD = 1536
ROWS = E * CPAD
F2 = F // 2

NCORES = 2
NSUB = 16
NW = NCORES * NSUB


def _router_body(x_ref, rw_ref, pos0_ref, pos1_ref, cw0_ref, cw1_ref, cnt_ref):
    x = x_ref[...]
    rw = rw_ref[...]
    logits = lax.dot_general(
        x, rw, (((1,), (1,)), ((), ())),
        preferred_element_type=jnp.float32)

    ilane = lax.broadcasted_iota(jnp.int32, (N, E), 1)
    m0 = jnp.max(logits, axis=1, keepdims=True)
    e0 = jnp.min(jnp.where(logits == m0, ilane, E), axis=1, keepdims=True)
    b0 = ilane == e0
    p2 = jnp.where(b0, -jnp.inf, logits)
    m1 = jnp.max(p2, axis=1, keepdims=True)
    e1 = jnp.min(jnp.where(p2 == m1, ilane, E), axis=1, keepdims=True)
    b1 = ilane == e1

    b0f = b0.astype(jnp.float32)
    b1f = b1.astype(jnp.float32)
    s = b0f + b1f

    r128 = lax.broadcasted_iota(jnp.int32, (128, 128), 0)
    c128 = lax.broadcasted_iota(jnp.int32, (128, 128), 1)
    tri = (c128 < r128).astype(jnp.float32)
    parts = []
    carry = jnp.zeros((1, E), jnp.float32)
    for b in range(N // 128):
        blk = s[b * 128:(b + 1) * 128]
        inner = lax.dot_general(tri, blk, (((1,), (0,)), ((), ())),
                                preferred_element_type=jnp.float32)
        parts.append(inner + carry)
        carry = carry + jnp.sum(blk, axis=0, keepdims=True)
    excl = jnp.concatenate(parts, axis=0)

    rank0 = jnp.sum(excl * b0f, axis=1, keepdims=True).astype(jnp.int32)
    rank1 = jnp.sum(excl * b1f, axis=1, keepdims=True).astype(jnp.int32)
    keep0 = rank0 < CAP
    keep1 = rank1 < CAP

    trash = CPAD - 1
    pos0 = e0 * CPAD + jnp.where(keep0, rank0, trash)
    pos1 = e1 * CPAD + jnp.where(keep1, rank1, trash)
    w0 = 1.0 / (1.0 + jnp.exp(m1 - m0))
    cw0 = jnp.where(keep0, w0, 0.0)
    cw1 = jnp.where(keep1, 1.0 - w0, 0.0)

    pos0_ref[...] = pos0
    pos1_ref[...] = pos1
    cw0_ref[...] = cw0
    cw1_ref[...] = cw1
    kept = jnp.minimum(carry, float(CAP)).astype(jnp.int32)
    cnt_ref[...] = jnp.broadcast_to(kept, (8, E))


def _router_call(x_flat, router_w):
    return pl.pallas_call(
        _router_body,
        out_shape=(
            jax.ShapeDtypeStruct((N, 1), jnp.int32),
            jax.ShapeDtypeStruct((N, 1), jnp.int32),
            jax.ShapeDtypeStruct((N, 1), jnp.float32),
            jax.ShapeDtypeStruct((N, 1), jnp.float32),
            jax.ShapeDtypeStruct((8, E), jnp.int32),
        ),
    )(x_flat, router_w)


@functools.lru_cache(maxsize=1)
def _sc_mesh():
    return plsc.VectorSubcoreMesh(core_axis_name="c", subcore_axis_name="s")


_DCHUNK = 32


def _dispatch_call(xf, posk):
    per_w = NK // NW

    @functools.partial(
        pl.kernel,
        out_type=jax.ShapeDtypeStruct((ROWS, H), jnp.float32),
        mesh=_sc_mesh(),
        scratch_types=[
            pltpu.VMEM((_DCHUNK,), jnp.int32),
            pltpu.VMEM((_DCHUNK, H), jnp.float32),
            pltpu.SemaphoreType.DMA,
        ],
    )
    def scatter_k(xf_hbm, pos_hbm, tok_hbm, idx_v, rows_v, sem):
        wid = lax.axis_index("s") * NCORES + lax.axis_index("c")
        base = wid * per_w

        @pl.loop(0, per_w, step=_DCHUNK)
        def _(off):
            s0 = base + off
            n0 = lax.rem(s0, N)
            pltpu.sync_copy(pos_hbm.at[pl.ds(s0, _DCHUNK)], idx_v)
            pltpu.sync_copy(xf_hbm.at[pl.ds(n0, _DCHUNK)], rows_v)
            pltpu.async_copy(rows_v, tok_hbm.at[idx_v], sem).wait()

    return scatter_k(xf, posk)


def _ffn_body(kept_ref, tok_ref, w1_ref, w2_ref, out_ref):
    e = pl.program_id(0)
    fh = pl.program_id(1)
    cb = pl.program_id(2)

    @pl.when(kept_ref[e] > cb * CB)
    def _():
        xb = tok_ref[...]
        h = lax.dot_general(xb, w1_ref[0], (((1,), (0,)), ((), ())),
                            preferred_element_type=jnp.float32)
        h = h / (1.0 + jnp.exp(-h))
        o = lax.dot_general(h, w2_ref[0], (((1,), (0,)), ((), ())),
                            preferred_element_type=jnp.float32)
        sl = pl.ds(cb * CB, CB)

        @pl.when(fh == 0)
        def _():
            out_ref[sl, :] = o

        @pl.when(fh > 0)
        def _():
            out_ref[sl, :] += o


def _ffn_call(kept, tok, w1, w2):
    nblk = CPAD // CB
    grid_spec = pltpu.PrefetchScalarGridSpec(
        num_scalar_prefetch=1,
        grid=(E, F // F2, nblk),
        in_specs=[
            pl.BlockSpec((CB, H), lambda e, fh, cb, k: (e * nblk + cb, 0)),
            pl.BlockSpec((1, H, F2), lambda e, fh, cb, k: (e, 0, fh)),
            pl.BlockSpec((1, F2, H), lambda e, fh, cb, k: (e, fh, 0)),
        ],
        out_specs=pl.BlockSpec((CPAD, H), lambda e, fh, cb, k: (e, 0)),
    )
    return pl.pallas_call(
        _ffn_body,
        grid_spec=grid_spec,
        out_shape=jax.ShapeDtypeStruct((ROWS, H), jnp.float32),
    )(kept, tok, w1, w2)


_GCHUNK = 32


def _gather_call(eo, posk):
    per_w = NK // NW

    @functools.partial(
        pl.kernel,
        out_type=jax.ShapeDtypeStruct((NK, H), jnp.float32),
        mesh=_sc_mesh(),
        scratch_types=[
            pltpu.VMEM((_GCHUNK,), jnp.int32),
            pltpu.VMEM((_GCHUNK, H), jnp.float32),
            pltpu.SemaphoreType.DMA,
        ],
    )
    def gather_k(eo_hbm, pos_hbm, g_hbm, idx_v, rows_v, sem):
        wid = lax.axis_index("s") * NCORES + lax.axis_index("c")
        base = wid * per_w

        @pl.loop(0, per_w, step=_GCHUNK)
        def _(off):
            s0 = base + off
            pltpu.sync_copy(pos_hbm.at[pl.ds(s0, _GCHUNK)], idx_v)
            pltpu.async_copy(eo_hbm.at[idx_v], rows_v, sem).wait()
            pltpu.sync_copy(rows_v, g_hbm.at[pl.ds(s0, _GCHUNK)])

    return gather_k(eo, posk)


_RB = 512


def _combine_body(x_ref, g0_ref, g1_ref, cw0_ref, cw1_ref, out_ref):
    cw0 = cw0_ref[...]
    cw1 = cw1_ref[...]
    t0 = jnp.where(cw0 > 0.0, g0_ref[...] * cw0, 0.0)
    t1 = jnp.where(cw1 > 0.0, g1_ref[...] * cw1, 0.0)
    out_ref[...] = x_ref[...] + t0 + t1


def _combine_call(x_flat, g, cw0, cw1):
    nblk = N // _RB
    return pl.pallas_call(
        _combine_body,
        grid=(nblk,),
        in_specs=[
            pl.BlockSpec((_RB, H), lambda i: (i, 0)),
            pl.BlockSpec((_RB, H), lambda i: (i, 0)),
            pl.BlockSpec((_RB, H), lambda i: (i + nblk, 0)),
            pl.BlockSpec((_RB, 1), lambda i: (i, 0)),
            pl.BlockSpec((_RB, 1), lambda i: (i, 0)),
        ],
        out_specs=pl.BlockSpec((_RB, H), lambda i: (i, 0)),
        out_shape=jax.ShapeDtypeStruct((N, H), jnp.float32),
    )(x_flat, g, g, cw0, cw1)


def kernel(x, router_w, w1, w2):
    B, T, _ = x.shape
    x_flat = x.reshape(N, H)

    pos0, pos1, cw0, cw1, cnt = _router_call(x_flat, router_w)
    posk = jnp.concatenate([pos0.reshape(N), pos1.reshape(N)])

    tok = _dispatch_call(x_flat, posk)
    eo = _ffn_call(cnt[0], tok, w1, w2)
    g = _gather_call(eo, posk)
    out = _combine_call(x_flat, g, cw0, cw1)
    return out.reshape(B, T, H)
